# Initial kernel scaffold; baseline (speedup 1.0000x reference)
import jax
import jax.numpy as jnp
from jax.experimental import pallas as pl


def kernel(x0, x1, x2, x3, x4, P0, P1, P2, P3, P4, e0, e1, e2, e3, comp0, bases0, root0, bias0, gam0, bet0, comp1, bases1, root1, bias1, gam1, bet1):
    raise NotImplementedError("write your pallas kernel here")


# speedup vs baseline: 1.0001x; 1.0001x over previous
import jax
import jax.numpy as jnp

_N = 10000
_SRC_OFF = [0, 0, 4000, 2000]
_DST_OFF = [2000, 4000, 6000, 8000]


def _layer(h, edges, comp, bases, root, bias):
    W = jnp.einsum('rb,bio->rio', comp, bases)
    out = h @ root + bias
    for r, (src, dst) in enumerate(edges):
        msg = h[src] @ W[r]
        s = jax.ops.segment_sum(msg, dst, num_segments=_N)
        cnt = jax.ops.segment_sum(jnp.ones(dst.shape[0], dtype=jnp.float32), dst, num_segments=_N)
        out = out + s / jnp.clip(cnt, 1.0)[:, None]
    return out


def _ln(h, g, b):
    mu = jnp.mean(h, axis=-1, keepdims=True)
    v = jnp.mean((h - mu) ** 2, axis=-1, keepdims=True)
    return (h - mu) / jnp.sqrt(v + 1e-5) * g + b


def kernel(x0, x1, x2, x3, x4, P0, P1, P2, P3, P4, e0, e1, e2, e3,
           comp0, bases0, root0, bias0, gam0, bet0,
           comp1, bases1, root1, bias1, gam1, bet1):
    h = jnp.concatenate([x0 @ P0, x1 @ P1, x2 @ P2, x3 @ P3, x4 @ P4], axis=0)
    es = [e0, e1, e2, e3]
    edges = [(es[r][0] + _SRC_OFF[r], es[r][1] + _DST_OFF[r]) for r in range(4)]
    for comp, bases, root, bias, g, b in [
        (comp0, bases0, root0, bias0, gam0, bet0),
        (comp1, bases1, root1, bias1, gam1, bet1)]:
        h = _layer(h, edges, comp, bases, root, bias)
        h = _ln(h, g, b)
        h = jax.nn.relu(h)
    return h


# speedup vs baseline: 3.4429x; 3.4426x over previous
import functools

import jax
import jax.numpy as jnp
from jax import lax
from jax.experimental import pallas as pl
from jax.experimental.pallas import tpu as pltpu
from jax.experimental.pallas import tpu_sc as plsc

NT = 2000
NTP = 2048
D_IN = 2048
H = 128
E = 80000
R = 4
SRC_BLK = [0, 0, 2, 1]

NC, NS = 2, 16
CH = 5120
E_PAD = NS * CH
NG = CH // 128
PROWS = 512
NPP = 2
SLAB = PROWS * NTP
STRIPE = SLAB // NS
ZCH = STRIPE // 8


def _sc_build_a(dst_ref, src_ref, a_ref, dstb, srcb, idxb, valb, zbuf, slab):
    c = lax.axis_index("c")
    s = lax.axis_index("s")
    zeros16 = jnp.zeros((16,), jnp.float32)

    @pl.loop(0, ZCH // 16)
    def _zero_zbuf(i):
        zbuf[pl.ds(i * 16, 16)] = zeros16

    for r in range(R):
        eb = r * E_PAD + s * CH
        pltpu.sync_copy(dst_ref.at[pl.ds(eb, CH)], dstb)
        pltpu.sync_copy(src_ref.at[pl.ds(eb, CH)], srcb)
        for pp in range(NPP):
            base = c * (NPP * PROWS) + pp * PROWS
            for j in range(8):
                pltpu.sync_copy(zbuf, slab.at[pl.ds(s * STRIPE + j * ZCH, ZCH)])

            @pl.loop(0, NG)
            def _compute(i):
                for j in range(8):
                    off = i * 128 + j * 16
                    dv = dstb[pl.ds(off, 16)]
                    sv = srcb[pl.ds(off, 16)]
                    m = (dv >= base) & (dv < base + PROWS)
                    f = (dv - base) * NTP + sv
                    idxb[i, pl.ds(j * 16, 16)] = jnp.where(m, f, 0)
                    valb[i, pl.ds(j * 16, 16)] = jnp.where(m, 1.0, 0.0)

            plsc.subcore_barrier()

            @pl.loop(0, NG)
            def _scatter(i):
                pltpu.sync_copy(valb.at[i], slab.at[idxb.at[i]], add=True)

            plsc.subcore_barrier()
            pltpu.sync_copy(
                slab.at[pl.ds(s * STRIPE, STRIPE)],
                a_ref.at[pl.ds(r * (NTP * NTP) + base * NTP + s * STRIPE, STRIPE)],
            )


def _build_a(e0, e1, e2, e3):
    dsts, srcs = [], []
    for e in (e0, e1, e2, e3):
        dsts.append(jnp.pad(e[1], (0, E_PAD - E), constant_values=2 * NT))
        srcs.append(jnp.pad(e[0], (0, E_PAD - E), constant_values=0))
    dst_flat = jnp.concatenate(dsts)
    src_flat = jnp.concatenate(srcs)
    mesh = plsc.VectorSubcoreMesh(
        core_axis_name="c", subcore_axis_name="s", num_cores=NC, num_subcores=NS
    )
    scatter = pl.kernel(
        _sc_build_a,
        out_type=jax.ShapeDtypeStruct((R * NTP * NTP,), jnp.float32),
        mesh=mesh,
        scratch_types=[
            pltpu.VMEM((CH,), jnp.int32),
            pltpu.VMEM((CH,), jnp.int32),
            pltpu.VMEM((NG, 128), jnp.int32),
            pltpu.VMEM((NG, 128), jnp.float32),
            pltpu.VMEM((ZCH,), jnp.float32),
            pltpu.VMEM_SHARED((SLAB,), jnp.float32),
        ],
    )
    return scatter(dst_flat, src_flat).reshape(R, NTP, NTP)


def _proj_body(x_ref, p_ref, o_ref):
    o_ref[...] = jnp.dot(x_ref[...], p_ref[...],
                         preferred_element_type=jnp.float32)


def _proj(x, p):
    return pl.pallas_call(
        _proj_body,
        grid=(5,),
        in_specs=[
            pl.BlockSpec((400, D_IN), lambda i: (i, 0)),
            pl.BlockSpec((D_IN, H), lambda i: (0, 0)),
        ],
        out_specs=pl.BlockSpec((400, H), lambda i: (i, 0)),
        out_shape=jax.ShapeDtypeStruct((NT, H), jnp.float32),
    )(x, p)


def _msg_body(a_ref, h_ref, o_ref):
    a = a_ref[0]
    hs = h_ref[...]
    hp = jnp.concatenate([hs, jnp.zeros((NTP - NT, H), jnp.float32)], axis=0)
    cnt = jnp.sum(a, axis=1)
    m = jnp.dot(a, hp, preferred_element_type=jnp.float32)
    o_ref[0] = m * (1.0 / jnp.maximum(cnt, 1.0))[:, None]


def _msg(a, h):
    def h_idx(r, mblk):
        sb = jnp.where(r == 2, 2, jnp.where(r == 3, 1, 0))
        return (sb, 0)

    return pl.pallas_call(
        _msg_body,
        grid=(R, NTP // 256),
        in_specs=[
            pl.BlockSpec((1, 256, NTP), lambda r, mblk: (r, mblk, 0)),
            pl.BlockSpec((NT, H), h_idx),
        ],
        out_specs=pl.BlockSpec((1, 256, H), lambda r, mblk: (r, mblk, 0)),
        out_shape=jax.ShapeDtypeStruct((R, NTP, H), jnp.float32),
    )(a, h)


def _comb_body(h_ref, msg_ref, comp_ref, bases_ref, root_ref, bias_ref,
               g_ref, b_ref, o_ref):
    blk = pl.program_id(0)
    r = blk - 1
    h = h_ref[...]
    out = jnp.dot(h, root_ref[...], preferred_element_type=jnp.float32)
    out = out + bias_ref[...][None, :]
    comp = comp_ref[...]
    sel = lax.broadcasted_iota(jnp.int32, (R, R), 0) == r
    cr = jnp.sum(jnp.where(sel, comp, 0.0), axis=0)
    bs = bases_ref[...]
    w = jnp.sum(bs * cr[:, None, None], axis=0)
    m = msg_ref[0][:NT]
    out = out + jnp.dot(m, w, preferred_element_type=jnp.float32)
    mu = jnp.mean(out, axis=1, keepdims=True)
    v = jnp.mean((out - mu) ** 2, axis=1, keepdims=True)
    y = (out - mu) / jnp.sqrt(v + 1e-5) * g_ref[...][None, :] + b_ref[...][None, :]
    o_ref[...] = jnp.maximum(y, 0.0)


def _combine(h, msg, comp, bases, root, bias, g, b):
    return pl.pallas_call(
        _comb_body,
        grid=(5,),
        in_specs=[
            pl.BlockSpec((NT, H), lambda blk: (blk, 0)),
            pl.BlockSpec((1, NTP, H), lambda blk: (jnp.maximum(blk - 1, 0), 0, 0)),
            pl.BlockSpec((R, R), lambda blk: (0, 0)),
            pl.BlockSpec((R, H, H), lambda blk: (0, 0, 0)),
            pl.BlockSpec((H, H), lambda blk: (0, 0)),
            pl.BlockSpec((H,), lambda blk: (0,)),
            pl.BlockSpec((H,), lambda blk: (0,)),
            pl.BlockSpec((H,), lambda blk: (0,)),
        ],
        out_specs=pl.BlockSpec((NT, H), lambda blk: (blk, 0)),
        out_shape=jax.ShapeDtypeStruct((5 * NT, H), jnp.float32),
    )(h, msg, comp, bases, root, bias, g, b)


def kernel(x0, x1, x2, x3, x4, P0, P1, P2, P3, P4, e0, e1, e2, e3,
           comp0, bases0, root0, bias0, gam0, bet0,
           comp1, bases1, root1, bias1, gam1, bet1):
    a = _build_a(e0, e1, e2, e3)
    h = jnp.concatenate([_proj(x0, P0), _proj(x1, P1), _proj(x2, P2),
                         _proj(x3, P3), _proj(x4, P4)], axis=0)
    for comp, bases, root, bias, g, b in [
        (comp0, bases0, root0, bias0, gam0, bet0),
        (comp1, bases1, root1, bias1, gam1, bet1),
    ]:
        msg = _msg(a, h)
        h = _combine(h, msg, comp, bases, root, bias, g, b)
    return h


# speedup vs baseline: 3.4699x; 1.0078x over previous
import functools

import jax
import jax.numpy as jnp
from jax import lax
from jax.experimental import pallas as pl
from jax.experimental.pallas import tpu as pltpu
from jax.experimental.pallas import tpu_sc as plsc

NT = 2000
NTP = 2048
D_IN = 2048
H = 128
E = 80000
R = 4
SRC_BLK = [0, 0, 2, 1]

NC, NS = 2, 16
CH = 5120
E_PAD = NS * CH
NG = CH // 128
PROWS = 512
NPP = 2
SLAB = PROWS * NTP
STRIPE = SLAB // NS
ZCH = STRIPE // 4


def _sc_build_a(dst_ref, src_ref, a_ref, dstb, srcb, idxb, valb, zbuf, slab, sem):
    c = lax.axis_index("c")
    s = lax.axis_index("s")
    zeros16 = jnp.zeros((16,), jnp.float32)

    @pl.loop(0, ZCH // 16)
    def _zero_zbuf(i):
        zbuf[pl.ds(i * 16, 16)] = zeros16

    for r in range(R):
        eb = r * E_PAD + s * CH
        pltpu.sync_copy(dst_ref.at[pl.ds(eb, CH)], dstb)
        pltpu.sync_copy(src_ref.at[pl.ds(eb, CH)], srcb)
        for pp in range(NPP):
            base = c * (NPP * PROWS) + pp * PROWS
            for j in range(4):
                pltpu.async_copy(zbuf, slab.at[pl.ds(s * STRIPE + j * ZCH, ZCH)], sem)

            @pl.loop(0, NG)
            def _compute(i):
                for j in range(8):
                    off = i * 128 + j * 16
                    dv = dstb[pl.ds(off, 16)]
                    sv = srcb[pl.ds(off, 16)]
                    m = (dv >= base) & (dv < base + PROWS)
                    f = (dv - base) * NTP + sv
                    idxb[i, pl.ds(j * 16, 16)] = jnp.where(m, f, 0)
                    valb[i, pl.ds(j * 16, 16)] = jnp.where(m, 1.0, 0.0)

            for j in range(4):
                pltpu.make_async_copy(
                    zbuf, slab.at[pl.ds(s * STRIPE + j * ZCH, ZCH)], sem).wait()

            plsc.subcore_barrier()

            @pl.loop(0, NG)
            def _scatter(i):
                pltpu.async_copy(valb.at[i], slab.at[idxb.at[i]], sem, add=True)

            @pl.loop(0, NG)
            def _drain(i):
                pltpu.make_async_copy(valb.at[i], slab.at[idxb.at[i]], sem).wait()

            plsc.subcore_barrier()
            pltpu.sync_copy(
                slab.at[pl.ds(s * STRIPE, STRIPE)],
                a_ref.at[pl.ds(r * (NTP * NTP) + base * NTP + s * STRIPE, STRIPE)],
            )


def _build_a(e0, e1, e2, e3):
    dsts, srcs = [], []
    for e in (e0, e1, e2, e3):
        dsts.append(jnp.pad(e[1], (0, E_PAD - E), constant_values=2 * NT))
        srcs.append(jnp.pad(e[0], (0, E_PAD - E), constant_values=0))
    dst_flat = jnp.concatenate(dsts)
    src_flat = jnp.concatenate(srcs)
    mesh = plsc.VectorSubcoreMesh(
        core_axis_name="c", subcore_axis_name="s", num_cores=NC, num_subcores=NS
    )
    scatter = pl.kernel(
        _sc_build_a,
        out_type=jax.ShapeDtypeStruct((R * NTP * NTP,), jnp.float32),
        mesh=mesh,
        scratch_types=[
            pltpu.VMEM((CH,), jnp.int32),
            pltpu.VMEM((CH,), jnp.int32),
            pltpu.VMEM((NG, 128), jnp.int32),
            pltpu.VMEM((NG, 128), jnp.float32),
            pltpu.VMEM((ZCH,), jnp.float32),
            pltpu.VMEM_SHARED((SLAB,), jnp.float32),
            pltpu.SemaphoreType.DMA,
        ],
    )
    return scatter(dst_flat, src_flat).reshape(R, NTP, NTP)


def _proj_body(x_ref, p_ref, o_ref):
    o_ref[...] = jnp.dot(x_ref[...], p_ref[...],
                         preferred_element_type=jnp.float32)


def _proj(x, p):
    return pl.pallas_call(
        _proj_body,
        grid=(5,),
        in_specs=[
            pl.BlockSpec((400, D_IN), lambda i: (i, 0)),
            pl.BlockSpec((D_IN, H), lambda i: (0, 0)),
        ],
        out_specs=pl.BlockSpec((400, H), lambda i: (i, 0)),
        out_shape=jax.ShapeDtypeStruct((NT, H), jnp.float32),
    )(x, p)


def _msg_body(a_ref, h_ref, o_ref):
    a = a_ref[0]
    hs = h_ref[...]
    hp = jnp.concatenate([hs, jnp.zeros((NTP - NT, H), jnp.float32)], axis=0)
    cnt = jnp.sum(a, axis=1)
    m = jnp.dot(a, hp, preferred_element_type=jnp.float32)
    o_ref[0] = m * (1.0 / jnp.maximum(cnt, 1.0))[:, None]


def _msg(a, h):
    def h_idx(r, mblk):
        sb = jnp.where(r == 2, 2, jnp.where(r == 3, 1, 0))
        return (sb, 0)

    return pl.pallas_call(
        _msg_body,
        grid=(R, NTP // 256),
        in_specs=[
            pl.BlockSpec((1, 256, NTP), lambda r, mblk: (r, mblk, 0)),
            pl.BlockSpec((NT, H), h_idx),
        ],
        out_specs=pl.BlockSpec((1, 256, H), lambda r, mblk: (r, mblk, 0)),
        out_shape=jax.ShapeDtypeStruct((R, NTP, H), jnp.float32),
    )(a, h)


def _comb_body(h_ref, msg_ref, comp_ref, bases_ref, root_ref, bias_ref,
               g_ref, b_ref, o_ref):
    blk = pl.program_id(0)
    r = blk - 1
    h = h_ref[...]
    out = jnp.dot(h, root_ref[...], preferred_element_type=jnp.float32)
    out = out + bias_ref[...][None, :]
    comp = comp_ref[...]
    sel = lax.broadcasted_iota(jnp.int32, (R, R), 0) == r
    cr = jnp.sum(jnp.where(sel, comp, 0.0), axis=0)
    bs = bases_ref[...]
    w = jnp.sum(bs * cr[:, None, None], axis=0)
    m = msg_ref[0][:NT]
    out = out + jnp.dot(m, w, preferred_element_type=jnp.float32)
    mu = jnp.mean(out, axis=1, keepdims=True)
    v = jnp.mean((out - mu) ** 2, axis=1, keepdims=True)
    y = (out - mu) / jnp.sqrt(v + 1e-5) * g_ref[...][None, :] + b_ref[...][None, :]
    o_ref[...] = jnp.maximum(y, 0.0)


def _combine(h, msg, comp, bases, root, bias, g, b):
    return pl.pallas_call(
        _comb_body,
        grid=(5,),
        in_specs=[
            pl.BlockSpec((NT, H), lambda blk: (blk, 0)),
            pl.BlockSpec((1, NTP, H), lambda blk: (jnp.maximum(blk - 1, 0), 0, 0)),
            pl.BlockSpec((R, R), lambda blk: (0, 0)),
            pl.BlockSpec((R, H, H), lambda blk: (0, 0, 0)),
            pl.BlockSpec((H, H), lambda blk: (0, 0)),
            pl.BlockSpec((H,), lambda blk: (0,)),
            pl.BlockSpec((H,), lambda blk: (0,)),
            pl.BlockSpec((H,), lambda blk: (0,)),
        ],
        out_specs=pl.BlockSpec((NT, H), lambda blk: (blk, 0)),
        out_shape=jax.ShapeDtypeStruct((5 * NT, H), jnp.float32),
    )(h, msg, comp, bases, root, bias, g, b)


def kernel(x0, x1, x2, x3, x4, P0, P1, P2, P3, P4, e0, e1, e2, e3,
           comp0, bases0, root0, bias0, gam0, bet0,
           comp1, bases1, root1, bias1, gam1, bet1):
    a = _build_a(e0, e1, e2, e3)
    h = jnp.concatenate([_proj(x0, P0), _proj(x1, P1), _proj(x2, P2),
                         _proj(x3, P3), _proj(x4, P4)], axis=0)
    for comp, bases, root, bias, g, b in [
        (comp0, bases0, root0, bias0, gam0, bet0),
        (comp1, bases1, root1, bias1, gam1, bet1),
    ]:
        msg = _msg(a, h)
        h = _combine(h, msg, comp, bases, root, bias, g, b)
    return h


# speedup vs baseline: 3.8891x; 1.1208x over previous
import functools

import jax
import jax.numpy as jnp
from jax import lax
from jax.experimental import pallas as pl
from jax.experimental.pallas import tpu as pltpu
from jax.experimental.pallas import tpu_sc as plsc

NT = 2000
NTP = 2048
D_IN = 2048
H = 128
E = 80000
R = 4
SRC_BLK = [0, 0, 2, 1]

NC, NS = 2, 16
NROW = 63
ACC = NROW * NTP
EC = 320
E_PAD = 81920
NCHUNK = E_PAD // EC


def _sc_build_a(dst_ref, src_ref, a_ref, dsta, srca, dstb, srcb, acc,
                sema, semb):
    c = lax.axis_index("c")
    s = lax.axis_index("s")
    w = c * NS + s
    row0 = w * NROW
    zeros16 = jnp.zeros((16,), jnp.float32)
    ones16 = jnp.ones((16,), jnp.float32)

    def _process(db, sb):
        for g in range(EC // 16):
            dv = db[pl.ds(g * 16, 16)]
            sv = sb[pl.ds(g * 16, 16)]
            m = (dv >= row0) & (dv < row0 + NROW)
            f = ((dv - row0) << 11) + sv
            f = jnp.where(m, f, 0)
            plsc.addupdate_scatter(acc, [f], ones16, mask=m)

    for r in range(R):
        @pl.loop(0, ACC // 16)
        def _zero(i):
            acc[pl.ds(i * 16, 16)] = zeros16

        eb = r * E_PAD
        pltpu.async_copy(dst_ref.at[pl.ds(eb, EC)], dsta, sema)
        pltpu.async_copy(src_ref.at[pl.ds(eb, EC)], srca, sema)
        pltpu.async_copy(dst_ref.at[pl.ds(eb + EC, EC)], dstb, semb)
        pltpu.async_copy(src_ref.at[pl.ds(eb + EC, EC)], srcb, semb)

        @pl.loop(0, NCHUNK // 2)
        def _chunks(g):
            pltpu.make_async_copy(dst_ref.at[pl.ds(eb, EC)], dsta, sema).wait()
            pltpu.make_async_copy(src_ref.at[pl.ds(eb, EC)], srca, sema).wait()
            _process(dsta, srca)

            @pl.when(g < NCHUNK // 2 - 1)
            def _refill_a():
                off = eb + (2 * g + 2) * EC
                pltpu.async_copy(dst_ref.at[pl.ds(off, EC)], dsta, sema)
                pltpu.async_copy(src_ref.at[pl.ds(off, EC)], srca, sema)

            pltpu.make_async_copy(dst_ref.at[pl.ds(eb, EC)], dstb, semb).wait()
            pltpu.make_async_copy(src_ref.at[pl.ds(eb, EC)], srcb, semb).wait()
            _process(dstb, srcb)

            @pl.when(g < NCHUNK // 2 - 1)
            def _refill_b():
                off = eb + (2 * g + 3) * EC
                pltpu.async_copy(dst_ref.at[pl.ds(off, EC)], dstb, semb)
                pltpu.async_copy(src_ref.at[pl.ds(off, EC)], srcb, semb)

        pltpu.sync_copy(acc, a_ref.at[pl.ds(r * (NTP * NTP) + row0 * NTP, ACC)])


def _build_a(e0, e1, e2, e3):
    dsts, srcs = [], []
    for e in (e0, e1, e2, e3):
        dsts.append(jnp.pad(e[1], (0, E_PAD - E), constant_values=2 * NT))
        srcs.append(jnp.pad(e[0], (0, E_PAD - E), constant_values=0))
    dst_flat = jnp.concatenate(dsts)
    src_flat = jnp.concatenate(srcs)
    mesh = plsc.VectorSubcoreMesh(
        core_axis_name="c", subcore_axis_name="s", num_cores=NC, num_subcores=NS
    )
    scatter = pl.kernel(
        _sc_build_a,
        out_type=jax.ShapeDtypeStruct((R * NTP * NTP,), jnp.float32),
        mesh=mesh,
        compiler_params=pltpu.CompilerParams(needs_layout_passes=False),
        scratch_types=[
            pltpu.VMEM((EC,), jnp.int32),
            pltpu.VMEM((EC,), jnp.int32),
            pltpu.VMEM((EC,), jnp.int32),
            pltpu.VMEM((EC,), jnp.int32),
            pltpu.VMEM((ACC,), jnp.float32),
            pltpu.SemaphoreType.DMA,
            pltpu.SemaphoreType.DMA,
        ],
    )
    return scatter(dst_flat, src_flat).reshape(R, NTP, NTP)


def _proj_body(x_ref, p_ref, o_ref):
    o_ref[...] = jnp.dot(x_ref[...], p_ref[...],
                         preferred_element_type=jnp.float32)


def _proj(x, p):
    return pl.pallas_call(
        _proj_body,
        grid=(5,),
        in_specs=[
            pl.BlockSpec((400, D_IN), lambda i: (i, 0)),
            pl.BlockSpec((D_IN, H), lambda i: (0, 0)),
        ],
        out_specs=pl.BlockSpec((400, H), lambda i: (i, 0)),
        out_shape=jax.ShapeDtypeStruct((NT, H), jnp.float32),
    )(x, p)


def _msg_body(a_ref, h_ref, o_ref):
    a = a_ref[0]
    hs = h_ref[...]
    hp = jnp.concatenate([hs, jnp.zeros((NTP - NT, H), jnp.float32)], axis=0)
    cnt = jnp.sum(a, axis=1)
    m = jnp.dot(a, hp, preferred_element_type=jnp.float32)
    o_ref[0] = m * (1.0 / jnp.maximum(cnt, 1.0))[:, None]


def _msg(a, h):
    def h_idx(r, mblk):
        sb = jnp.where(r == 2, 2, jnp.where(r == 3, 1, 0))
        return (sb, 0)

    return pl.pallas_call(
        _msg_body,
        grid=(R, NTP // 256),
        in_specs=[
            pl.BlockSpec((1, 256, NTP), lambda r, mblk: (r, mblk, 0)),
            pl.BlockSpec((NT, H), h_idx),
        ],
        out_specs=pl.BlockSpec((1, 256, H), lambda r, mblk: (r, mblk, 0)),
        out_shape=jax.ShapeDtypeStruct((R, NTP, H), jnp.float32),
    )(a, h)


def _comb_body(h_ref, msg_ref, comp_ref, bases_ref, root_ref, bias_ref,
               g_ref, b_ref, o_ref):
    blk = pl.program_id(0)
    r = blk - 1
    h = h_ref[...]
    out = jnp.dot(h, root_ref[...], preferred_element_type=jnp.float32)
    out = out + bias_ref[...][None, :]
    comp = comp_ref[...]
    sel = lax.broadcasted_iota(jnp.int32, (R, R), 0) == r
    cr = jnp.sum(jnp.where(sel, comp, 0.0), axis=0)
    bs = bases_ref[...]
    w = jnp.sum(bs * cr[:, None, None], axis=0)
    m = msg_ref[0][:NT]
    out = out + jnp.dot(m, w, preferred_element_type=jnp.float32)
    mu = jnp.mean(out, axis=1, keepdims=True)
    v = jnp.mean((out - mu) ** 2, axis=1, keepdims=True)
    y = (out - mu) / jnp.sqrt(v + 1e-5) * g_ref[...][None, :] + b_ref[...][None, :]
    o_ref[...] = jnp.maximum(y, 0.0)


def _combine(h, msg, comp, bases, root, bias, g, b):
    return pl.pallas_call(
        _comb_body,
        grid=(5,),
        in_specs=[
            pl.BlockSpec((NT, H), lambda blk: (blk, 0)),
            pl.BlockSpec((1, NTP, H), lambda blk: (jnp.maximum(blk - 1, 0), 0, 0)),
            pl.BlockSpec((R, R), lambda blk: (0, 0)),
            pl.BlockSpec((R, H, H), lambda blk: (0, 0, 0)),
            pl.BlockSpec((H, H), lambda blk: (0, 0)),
            pl.BlockSpec((H,), lambda blk: (0,)),
            pl.BlockSpec((H,), lambda blk: (0,)),
            pl.BlockSpec((H,), lambda blk: (0,)),
        ],
        out_specs=pl.BlockSpec((NT, H), lambda blk: (blk, 0)),
        out_shape=jax.ShapeDtypeStruct((5 * NT, H), jnp.float32),
    )(h, msg, comp, bases, root, bias, g, b)


def kernel(x0, x1, x2, x3, x4, P0, P1, P2, P3, P4, e0, e1, e2, e3,
           comp0, bases0, root0, bias0, gam0, bet0,
           comp1, bases1, root1, bias1, gam1, bet1):
    a = _build_a(e0, e1, e2, e3)
    h = jnp.concatenate([_proj(x0, P0), _proj(x1, P1), _proj(x2, P2),
                         _proj(x3, P3), _proj(x4, P4)], axis=0)
    for comp, bases, root, bias, g, b in [
        (comp0, bases0, root0, bias0, gam0, bet0),
        (comp1, bases1, root1, bias1, gam1, bet1),
    ]:
        msg = _msg(a, h)
        h = _combine(h, msg, comp, bases, root, bias, g, b)
    return h
